# baseline (speedup 1.0000x reference)
import functools

import jax
import jax.numpy as jnp
from jax import lax
from jax.experimental import pallas as pl
from jax.experimental.pallas import tpu as pltpu
from jax.experimental.pallas import tpu_sc as plsc

_EPS = 1e-5
_L = 16


def _deg_body(NP, EW, NC, src_hbm, dst_hbm, ho_hbm, hi_hbm, idx_v, hist_o, hist_i):
    wid = lax.axis_index("s") * NC + lax.axis_index("c")
    zeros = jnp.zeros((_L,), jnp.float32)
    ones = jnp.full((_L,), 1.0, jnp.float32)

    def zero_body(i, carry):
        hist_o[pl.ds(i * _L, _L)] = zeros
        hist_i[pl.ds(i * _L, _L)] = zeros
        return carry

    lax.fori_loop(0, NP // _L, zero_body, 0)

    base = wid * EW
    pltpu.sync_copy(src_hbm.at[pl.ds(base, EW)], idx_v)

    def add_o(i, carry):
        plsc.addupdate_scatter(hist_o, [idx_v[pl.ds(i * _L, _L)]], ones)
        return carry

    lax.fori_loop(0, EW // _L, add_o, 0)

    pltpu.sync_copy(dst_hbm.at[pl.ds(base, EW)], idx_v)

    def add_i(i, carry):
        plsc.addupdate_scatter(hist_i, [idx_v[pl.ds(i * _L, _L)]], ones)
        return carry

    lax.fori_loop(0, EW // _L, add_i, 0)

    pltpu.sync_copy(hist_o, ho_hbm.at[wid])
    pltpu.sync_copy(hist_i, hi_hbm.at[wid])


def _proj_body(x_ref, w1c_ref, ho_ref, hi_ref, t_ref, no_ref, ni_ref):
    t_ref[...] = jnp.dot(x_ref[...], w1c_ref[...],
                         preferred_element_type=jnp.float32)
    no_ref[...] = lax.rsqrt(jnp.maximum(jnp.sum(ho_ref[...], axis=0), 1.0))
    ni_ref[...] = lax.rsqrt(jnp.maximum(jnp.sum(hi_ref[...], axis=0), 1.0))


def _gather_body(NP, EW, NC, CH, H,
                 t_hbm, no_hbm, ni_hbm, src_hbm, dst_hbm,
                 g_hbm, en_hbm,
                 sidx, didx, no_tab, ni_tab,
                 ts0, td0, ts1, td1, gb0, gb1, enbuf,
                 sem0, sem1, sems0, sems1):
    wid = lax.axis_index("s") * NC + lax.axis_index("c")
    base = wid * EW
    pltpu.sync_copy(no_hbm, no_tab)
    pltpu.sync_copy(ni_hbm, ni_tab)
    pltpu.sync_copy(src_hbm.at[pl.ds(base, EW)], sidx.at[pl.ds(0, EW)])
    pltpu.sync_copy(dst_hbm.at[pl.ds(base, EW)], didx.at[pl.ds(0, EW)])
    zeros16 = jnp.zeros((_L,), jnp.int32)
    sidx[pl.ds(EW, _L)] = zeros16
    didx[pl.ds(EW, _L)] = zeros16

    nfull = EW // CH
    last = nfull - 1

    def issue(c, ts, td, sem):
        off = c * CH
        pltpu.async_copy(t_hbm.at[sidx.at[pl.ds(off, CH)]], ts, sem)
        pltpu.async_copy(t_hbm.at[didx.at[pl.ds(off, CH)]], td, sem)

    def process(c, ts, td, gb, sem, sems, wait_store):
        off = c * CH
        pltpu.make_async_copy(t_hbm.at[sidx.at[pl.ds(off, CH)]], ts, sem).wait()
        pltpu.make_async_copy(t_hbm.at[didx.at[pl.ds(off, CH)]], td, sem).wait()
        if wait_store:
            pltpu.make_async_copy(gb, g_hbm.at[pl.ds(base + off, CH)],
                                  sems).wait()

        def row(i, carry):
            for k in range(H // _L):
                sl = pl.ds(k * _L, _L)
                sh = pl.ds(H + k * _L, _L)
                gb[i, sl] = ts[i, sl]
                gb[i, sh] = td[i, sh]
            return carry

        lax.fori_loop(0, CH, row, 0)

        for k in range(CH // _L):
            s16 = sidx[pl.ds(off + k * _L, _L)]
            d16 = didx[pl.ds(off + k * _L, _L)]
            en16 = plsc.load_gather(no_tab, [s16]) * plsc.load_gather(ni_tab, [d16])
            enbuf[pl.ds(off + k * _L, _L)] = en16

        pltpu.async_copy(gb, g_hbm.at[pl.ds(base + off, CH)], sems)

    issue(0, ts0, td0, sem0)
    issue(1, ts1, td1, sem1)
    process(0, ts0, td0, gb0, sem0, sems0, False)
    issue(2, ts0, td0, sem0)
    process(1, ts1, td1, gb1, sem1, sems1, False)

    npair = (nfull + 1) // 2 + 1

    def pair(cc, carry):
        c1 = jnp.minimum(2 * cc + 1, last)
        c2 = jnp.minimum(2 * cc + 2, last)
        c0 = jnp.minimum(2 * cc, last)
        issue(c1, ts1, td1, sem1)
        process(c0, ts0, td0, gb0, sem0, sems0, True)
        issue(c2, ts0, td0, sem0)
        process(c1, ts1, td1, gb1, sem1, sems1, True)
        return carry

    lax.fori_loop(1, npair, pair, 0)
    process(last, ts0, td0, gb0, sem0, sems0, True)
    pltpu.make_async_copy(gb0, g_hbm.at[pl.ds(base + last * CH, CH)],
                          sems0).wait()
    pltpu.make_async_copy(gb1, g_hbm.at[pl.ds(base + last * CH, CH)],
                          sems1).wait()

    tail = EW - nfull * CH
    if tail:
        toff = nfull * CH
        pltpu.async_copy(t_hbm.at[sidx.at[pl.ds(toff, tail)]],
                         ts0.at[pl.ds(0, tail)], sem0).wait()
        pltpu.async_copy(t_hbm.at[didx.at[pl.ds(toff, tail)]],
                         td0.at[pl.ds(0, tail)], sem0).wait()

        def trow(i, carry):
            for k in range(H // _L):
                sl = pl.ds(k * _L, _L)
                sh = pl.ds(H + k * _L, _L)
                gb0[i, sl] = ts0[i, sl]
                gb0[i, sh] = td0[i, sh]
            return carry

        lax.fori_loop(0, tail, trow, 0)
        for k in range(-(-tail // _L)):
            s16 = sidx[pl.ds(toff + k * _L, _L)]
            d16 = didx[pl.ds(toff + k * _L, _L)]
            en16 = plsc.load_gather(no_tab, [s16]) * plsc.load_gather(ni_tab, [d16])
            enbuf[pl.ds(toff + k * _L, _L)] = en16
        pltpu.sync_copy(gb0.at[pl.ds(0, tail)],
                        g_hbm.at[pl.ds(base + toff, tail)])

    pltpu.sync_copy(enbuf.at[pl.ds(0, EW)], en_hbm.at[pl.ds(base, EW)])


def _tdot(a, b):
    return lax.dot_general(a, b, (((0,), (0,)), ((), ())),
                           preferred_element_type=jnp.float32)


def _mlp_body(g_ref, gate_ref, en_ref, wc_ref, b1c_ref, vv1_ref, g1_ref,
              be1_ref, w2c_ref, b2c_ref, vv2_ref, g2_ref, be2_ref, w3_ref,
              b3_ref, adj_ref):
    g = g_ref[...]
    c = lax.dot_general(wc_ref[...], g, (((0,), (1,)), ((), ())),
                        preferred_element_type=jnp.float32) + b1c_ref[...]
    v = _tdot(vv1_ref[...], c * c)
    h = c * lax.rsqrt(v + _EPS) * g1_ref[...] + be1_ref[...]
    h = jnp.maximum(h, 0.0)
    c2 = _tdot(w2c_ref[...], h) + b2c_ref[...]
    v2 = _tdot(vv2_ref[...], c2 * c2)
    h2 = c2 * lax.rsqrt(v2 + _EPS) * g2_ref[...] + be2_ref[...]
    h2 = jnp.maximum(h2, 0.0)
    w = _tdot(w3_ref[...], h2) + b3_ref[...]
    adj_ref[...] = jax.nn.sigmoid(gate_ref[...] + w) * en_ref[...]


def kernel(node_embeddings, edge_index, W1, b1, g1, be1, W2, b2, g2, be2, W3, b3):
    N, D = node_embeddings.shape
    E = edge_index.shape[1]
    H = W1.shape[1]
    H2 = W2.shape[1]
    info = plsc.get_sparse_core_info()
    NC, NS = info.num_cores, info.num_subcores
    NW = NC * NS
    EW = E // NW
    RB = 1024
    NP = -(-N // RB) * RB
    CH = 96

    src = edge_index[0]
    dst = edge_index[1]

    noise = jax.random.uniform(jax.random.key(42), (E,), dtype=jnp.float32,
                               minval=1e-6, maxval=1.0 - 1e-6)
    gate = jnp.log(noise) - jnp.log(1.0 - noise)

    W1c = jnp.concatenate([W1[:D], W1[D:]], axis=1)
    C1 = jnp.eye(H, dtype=jnp.float32) - jnp.full((H, H), 1.0 / H, jnp.float32)
    Wc = jnp.concatenate([C1, C1], axis=0)
    b1c = (b1 - jnp.mean(b1)).reshape(1, H)
    vv1 = jnp.full((H, 1), 1.0 / H, jnp.float32)
    C2 = jnp.eye(H2, dtype=jnp.float32) - jnp.full((H2, H2), 1.0 / H2, jnp.float32)
    W2c = W2 @ C2
    b2c = (b2 - jnp.mean(b2)).reshape(1, H2)
    vv2 = jnp.full((H2, 1), 1.0 / H2, jnp.float32)

    mesh = plsc.VectorSubcoreMesh(core_axis_name="c", subcore_axis_name="s")
    ho, hi = pl.kernel(
        functools.partial(_deg_body, NP, EW, NC),
        out_type=[jax.ShapeDtypeStruct((NW, NP), jnp.float32),
                  jax.ShapeDtypeStruct((NW, NP), jnp.float32)],
        mesh=mesh,
        scratch_types=[pltpu.VMEM((EW,), jnp.int32),
                       pltpu.VMEM((NP,), jnp.float32),
                       pltpu.VMEM((NP,), jnp.float32)],
        compiler_params=pltpu.CompilerParams(needs_layout_passes=False),
    )(src, dst)

    t_tab, no2, ni2 = pl.pallas_call(
        _proj_body,
        grid=(NP // RB,),
        in_specs=[
            pl.BlockSpec((RB, D), lambda i: (i, 0)),
            pl.BlockSpec((D, 2 * H), lambda i: (0, 0)),
            pl.BlockSpec((NW, RB // 128, 128), lambda i: (0, i, 0)),
            pl.BlockSpec((NW, RB // 128, 128), lambda i: (0, i, 0)),
        ],
        out_specs=[
            pl.BlockSpec((RB, 2 * H), lambda i: (i, 0)),
            pl.BlockSpec((RB // 128, 128), lambda i: (i, 0)),
            pl.BlockSpec((RB // 128, 128), lambda i: (i, 0)),
        ],
        out_shape=[jax.ShapeDtypeStruct((NP, 2 * H), jnp.float32),
                   jax.ShapeDtypeStruct((NP // 128, 128), jnp.float32),
                   jax.ShapeDtypeStruct((NP // 128, 128), jnp.float32)],
    )(node_embeddings, W1c,
      ho.reshape(NW, NP // 128, 128), hi.reshape(NW, NP // 128, 128))

    no_flat = no2.reshape(NP)
    ni_flat = ni2.reshape(NP)
    BE = 3200

    def run_half(src_h, dst_h, gate_h):
        Eh = src_h.shape[0]
        EWh = Eh // NW
        g_rows, en = pl.kernel(
            functools.partial(_gather_body, NP, EWh, NC, CH, H),
            out_type=[jax.ShapeDtypeStruct((Eh, 2 * H), jnp.float32),
                      jax.ShapeDtypeStruct((Eh,), jnp.float32)],
            mesh=mesh,
            scratch_types=[pltpu.VMEM((EWh + _L,), jnp.int32),
                           pltpu.VMEM((EWh + _L,), jnp.int32),
                           pltpu.VMEM((NP,), jnp.float32),
                           pltpu.VMEM((NP,), jnp.float32),
                           pltpu.VMEM((CH, 2 * H), jnp.float32),
                           pltpu.VMEM((CH, 2 * H), jnp.float32),
                           pltpu.VMEM((CH, 2 * H), jnp.float32),
                           pltpu.VMEM((CH, 2 * H), jnp.float32),
                           pltpu.VMEM((CH, 2 * H), jnp.float32),
                           pltpu.VMEM((CH, 2 * H), jnp.float32),
                           pltpu.VMEM((EWh + _L,), jnp.float32),
                           pltpu.SemaphoreType.DMA,
                           pltpu.SemaphoreType.DMA,
                           pltpu.SemaphoreType.DMA,
                           pltpu.SemaphoreType.DMA],
            compiler_params=pltpu.CompilerParams(needs_layout_passes=False),
        )(t_tab, no_flat, ni_flat, src_h, dst_h)

        adj_h = pl.pallas_call(
            _mlp_body,
            grid=(Eh // BE,),
            in_specs=[
                pl.BlockSpec((BE, 2 * H), lambda i: (i, 0)),
                pl.BlockSpec((1, BE), lambda i: (0, i)),
                pl.BlockSpec((1, BE), lambda i: (0, i)),
                pl.BlockSpec((2 * H, H), lambda i: (0, 0)),
                pl.BlockSpec((H, 1), lambda i: (0, 0)),
                pl.BlockSpec((H, 1), lambda i: (0, 0)),
                pl.BlockSpec((H, 1), lambda i: (0, 0)),
                pl.BlockSpec((H, 1), lambda i: (0, 0)),
                pl.BlockSpec((H, H2), lambda i: (0, 0)),
                pl.BlockSpec((H2, 1), lambda i: (0, 0)),
                pl.BlockSpec((H2, 1), lambda i: (0, 0)),
                pl.BlockSpec((H2, 1), lambda i: (0, 0)),
                pl.BlockSpec((H2, 1), lambda i: (0, 0)),
                pl.BlockSpec((H2, 1), lambda i: (0, 0)),
                pl.BlockSpec((1, 1), lambda i: (0, 0)),
            ],
            out_specs=pl.BlockSpec((1, BE), lambda i: (0, i)),
            out_shape=jax.ShapeDtypeStruct((1, Eh), jnp.float32),
        )(g_rows, gate_h.reshape(1, Eh), en.reshape(1, Eh),
          Wc, b1c.reshape(H, 1), vv1, g1.reshape(H, 1), be1.reshape(H, 1),
          W2c, b2c.reshape(H2, 1), vv2, g2.reshape(H2, 1), be2.reshape(H2, 1),
          W3, b3.reshape(1, 1))
        return adj_h.reshape(Eh)

    half = E // 2
    adj1 = run_half(src[:half], dst[:half], gate[:half])
    adj2 = run_half(src[half:], dst[half:], gate[half:])
    return jnp.concatenate([adj1, adj2])


# speedup vs baseline: 1.0461x; 1.0461x over previous
import functools

import jax
import jax.numpy as jnp
from jax import lax
from jax.experimental import pallas as pl
from jax.experimental.pallas import tpu as pltpu
from jax.experimental.pallas import tpu_sc as plsc

_EPS = 1e-5
_L = 16


def _deg_body(NP, EW, NC, src_hbm, dst_hbm, ho_hbm, hi_hbm, idx_v, hist_o, hist_i):
    wid = lax.axis_index("s") * NC + lax.axis_index("c")
    zeros = jnp.zeros((_L,), jnp.float32)
    ones = jnp.full((_L,), 1.0, jnp.float32)

    def zero_body(i, carry):
        hist_o[pl.ds(i * _L, _L)] = zeros
        hist_i[pl.ds(i * _L, _L)] = zeros
        return carry

    lax.fori_loop(0, NP // _L, zero_body, 0)

    base = wid * EW
    pltpu.sync_copy(src_hbm.at[pl.ds(base, EW)], idx_v)

    def add_o(i, carry):
        plsc.addupdate_scatter(hist_o, [idx_v[pl.ds(i * _L, _L)]], ones)
        return carry

    lax.fori_loop(0, EW // _L, add_o, 0)

    pltpu.sync_copy(dst_hbm.at[pl.ds(base, EW)], idx_v)

    def add_i(i, carry):
        plsc.addupdate_scatter(hist_i, [idx_v[pl.ds(i * _L, _L)]], ones)
        return carry

    lax.fori_loop(0, EW // _L, add_i, 0)

    pltpu.sync_copy(hist_o, ho_hbm.at[wid])
    pltpu.sync_copy(hist_i, hi_hbm.at[wid])


def _proj_body(x_ref, w1c_ref, ho_ref, hi_ref, t_ref, no_ref, ni_ref):
    t_ref[...] = jnp.dot(x_ref[...], w1c_ref[...],
                         preferred_element_type=jnp.float32)
    no_ref[...] = lax.rsqrt(jnp.maximum(jnp.sum(ho_ref[...], axis=0), 1.0))
    ni_ref[...] = lax.rsqrt(jnp.maximum(jnp.sum(hi_ref[...], axis=0), 1.0))


def _gather_body(NP, EW, NC, CH, H,
                 t_hbm, no_hbm, ni_hbm, src_hbm, dst_hbm,
                 g_hbm, en_hbm,
                 sidx, didx, no_tab, ni_tab,
                 ts0, td0, ts1, td1, gb0, gb1, enbuf,
                 sem0, sem1, sems0, sems1):
    wid = lax.axis_index("s") * NC + lax.axis_index("c")
    base = wid * EW
    pltpu.sync_copy(no_hbm, no_tab)
    pltpu.sync_copy(ni_hbm, ni_tab)
    pltpu.sync_copy(src_hbm.at[pl.ds(base, EW)], sidx.at[pl.ds(0, EW)])
    pltpu.sync_copy(dst_hbm.at[pl.ds(base, EW)], didx.at[pl.ds(0, EW)])
    zeros16 = jnp.zeros((_L,), jnp.int32)
    sidx[pl.ds(EW, _L)] = zeros16
    didx[pl.ds(EW, _L)] = zeros16

    nfull = EW // CH
    last = nfull - 1

    def issue(c, ts, td, sem):
        off = c * CH
        pltpu.async_copy(t_hbm.at[sidx.at[pl.ds(off, CH)]], ts, sem)
        pltpu.async_copy(t_hbm.at[didx.at[pl.ds(off, CH)]], td, sem)

    def process(c, ts, td, gb, sem, sems, wait_store):
        off = c * CH
        pltpu.make_async_copy(t_hbm.at[sidx.at[pl.ds(off, CH)]], ts, sem).wait()
        pltpu.make_async_copy(t_hbm.at[didx.at[pl.ds(off, CH)]], td, sem).wait()
        if wait_store:
            pltpu.make_async_copy(gb, g_hbm.at[pl.ds(base + off, CH)],
                                  sems).wait()

        def row(i, carry):
            for k in range(H // _L):
                sl = pl.ds(k * _L, _L)
                sh = pl.ds(H + k * _L, _L)
                gb[i, sl] = ts[i, sl]
                gb[i, sh] = td[i, sh]
            return carry

        lax.fori_loop(0, CH, row, 0)

        for k in range(CH // _L):
            s16 = sidx[pl.ds(off + k * _L, _L)]
            d16 = didx[pl.ds(off + k * _L, _L)]
            en16 = plsc.load_gather(no_tab, [s16]) * plsc.load_gather(ni_tab, [d16])
            enbuf[pl.ds(off + k * _L, _L)] = en16

        pltpu.async_copy(gb, g_hbm.at[pl.ds(base + off, CH)], sems)

    issue(0, ts0, td0, sem0)
    issue(1, ts1, td1, sem1)
    process(0, ts0, td0, gb0, sem0, sems0, False)
    issue(2, ts0, td0, sem0)
    process(1, ts1, td1, gb1, sem1, sems1, False)

    npair = (nfull + 1) // 2 + 1

    def pair(cc, carry):
        c1 = jnp.minimum(2 * cc + 1, last)
        c2 = jnp.minimum(2 * cc + 2, last)
        c0 = jnp.minimum(2 * cc, last)
        issue(c1, ts1, td1, sem1)
        process(c0, ts0, td0, gb0, sem0, sems0, True)
        issue(c2, ts0, td0, sem0)
        process(c1, ts1, td1, gb1, sem1, sems1, True)
        return carry

    lax.fori_loop(1, npair, pair, 0)
    process(last, ts0, td0, gb0, sem0, sems0, True)
    pltpu.make_async_copy(gb0, g_hbm.at[pl.ds(base + last * CH, CH)],
                          sems0).wait()
    pltpu.make_async_copy(gb1, g_hbm.at[pl.ds(base + last * CH, CH)],
                          sems1).wait()

    tail = EW - nfull * CH
    if tail:
        toff = nfull * CH
        pltpu.async_copy(t_hbm.at[sidx.at[pl.ds(toff, tail)]],
                         ts0.at[pl.ds(0, tail)], sem0).wait()
        pltpu.async_copy(t_hbm.at[didx.at[pl.ds(toff, tail)]],
                         td0.at[pl.ds(0, tail)], sem0).wait()

        def trow(i, carry):
            for k in range(H // _L):
                sl = pl.ds(k * _L, _L)
                sh = pl.ds(H + k * _L, _L)
                gb0[i, sl] = ts0[i, sl]
                gb0[i, sh] = td0[i, sh]
            return carry

        lax.fori_loop(0, tail, trow, 0)
        for k in range(-(-tail // _L)):
            s16 = sidx[pl.ds(toff + k * _L, _L)]
            d16 = didx[pl.ds(toff + k * _L, _L)]
            en16 = plsc.load_gather(no_tab, [s16]) * plsc.load_gather(ni_tab, [d16])
            enbuf[pl.ds(toff + k * _L, _L)] = en16
        pltpu.sync_copy(gb0.at[pl.ds(0, tail)],
                        g_hbm.at[pl.ds(base + toff, tail)])

    pltpu.sync_copy(enbuf.at[pl.ds(0, EW)], en_hbm.at[pl.ds(base, EW)])


def _tdot(a, b):
    return lax.dot_general(a, b, (((0,), (0,)), ((), ())),
                           preferred_element_type=jnp.float32)


def _mlp_body(g_ref, gate_ref, en_ref, wc_ref, b1c_ref, vv1_ref, g1_ref,
              be1_ref, w2c_ref, b2c_ref, vv2_ref, g2_ref, be2_ref, w3_ref,
              b3_ref, adj_ref):
    g = g_ref[...]
    c = lax.dot_general(wc_ref[...], g, (((0,), (1,)), ((), ())),
                        preferred_element_type=jnp.float32) + b1c_ref[...]
    v = _tdot(vv1_ref[...], c * c)
    h = c * lax.rsqrt(v + _EPS) * g1_ref[...] + be1_ref[...]
    h = jnp.maximum(h, 0.0)
    c2 = _tdot(w2c_ref[...], h) + b2c_ref[...]
    v2 = _tdot(vv2_ref[...], c2 * c2)
    h2 = c2 * lax.rsqrt(v2 + _EPS) * g2_ref[...] + be2_ref[...]
    h2 = jnp.maximum(h2, 0.0)
    w = _tdot(w3_ref[...], h2) + b3_ref[...]
    adj_ref[...] = jax.nn.sigmoid(gate_ref[...] + w) * en_ref[...]


def kernel(node_embeddings, edge_index, W1, b1, g1, be1, W2, b2, g2, be2, W3, b3):
    N, D = node_embeddings.shape
    E = edge_index.shape[1]
    H = W1.shape[1]
    H2 = W2.shape[1]
    info = plsc.get_sparse_core_info()
    NC, NS = info.num_cores, info.num_subcores
    NW = NC * NS
    EW = E // NW
    RB = 1024
    NP = -(-N // RB) * RB
    CH = 96

    noise = jax.random.uniform(jax.random.key(42), (E,), dtype=jnp.float32,
                               minval=1e-6, maxval=1.0 - 1e-6)
    gate = jnp.log(noise) - jnp.log(1.0 - noise)

    W1c = jnp.concatenate([W1[:D], W1[D:]], axis=1)
    C1 = jnp.eye(H, dtype=jnp.float32) - jnp.full((H, H), 1.0 / H, jnp.float32)
    Wc = jnp.concatenate([C1, C1], axis=0)
    b1c = (b1 - jnp.mean(b1)).reshape(1, H)
    vv1 = jnp.full((H, 1), 1.0 / H, jnp.float32)
    C2 = jnp.eye(H2, dtype=jnp.float32) - jnp.full((H2, H2), 1.0 / H2, jnp.float32)
    W2c = W2 @ C2
    b2c = (b2 - jnp.mean(b2)).reshape(1, H2)
    vv2 = jnp.full((H2, 1), 1.0 / H2, jnp.float32)

    mesh = plsc.VectorSubcoreMesh(core_axis_name="c", subcore_axis_name="s")
    ho, hi = pl.kernel(
        functools.partial(_deg_body, NP, EW, NC),
        out_type=[jax.ShapeDtypeStruct((NW, NP), jnp.float32),
                  jax.ShapeDtypeStruct((NW, NP), jnp.float32)],
        mesh=mesh,
        scratch_types=[pltpu.VMEM((EW,), jnp.int32),
                       pltpu.VMEM((NP,), jnp.float32),
                       pltpu.VMEM((NP,), jnp.float32)],
        compiler_params=pltpu.CompilerParams(needs_layout_passes=False),
    )(edge_index[0], edge_index[1])

    t_tab, no2, ni2 = pl.pallas_call(
        _proj_body,
        grid=(NP // RB,),
        in_specs=[
            pl.BlockSpec((RB, D), lambda i: (i, 0)),
            pl.BlockSpec((D, 2 * H), lambda i: (0, 0)),
            pl.BlockSpec((NW, RB // 128, 128), lambda i: (0, i, 0)),
            pl.BlockSpec((NW, RB // 128, 128), lambda i: (0, i, 0)),
        ],
        out_specs=[
            pl.BlockSpec((RB, 2 * H), lambda i: (i, 0)),
            pl.BlockSpec((RB // 128, 128), lambda i: (i, 0)),
            pl.BlockSpec((RB // 128, 128), lambda i: (i, 0)),
        ],
        out_shape=[jax.ShapeDtypeStruct((NP, 2 * H), jnp.float32),
                   jax.ShapeDtypeStruct((NP // 128, 128), jnp.float32),
                   jax.ShapeDtypeStruct((NP // 128, 128), jnp.float32)],
    )(node_embeddings, W1c,
      ho.reshape(NW, NP // 128, 128), hi.reshape(NW, NP // 128, 128))

    g_rows, en = pl.kernel(
        functools.partial(_gather_body, NP, EW, NC, CH, H),
        out_type=[jax.ShapeDtypeStruct((E, 2 * H), jnp.float32),
                  jax.ShapeDtypeStruct((E,), jnp.float32)],
        mesh=mesh,
        scratch_types=[pltpu.VMEM((EW + _L,), jnp.int32),
                       pltpu.VMEM((EW + _L,), jnp.int32),
                       pltpu.VMEM((NP,), jnp.float32),
                       pltpu.VMEM((NP,), jnp.float32),
                       pltpu.VMEM((CH, 2 * H), jnp.float32),
                       pltpu.VMEM((CH, 2 * H), jnp.float32),
                       pltpu.VMEM((CH, 2 * H), jnp.float32),
                       pltpu.VMEM((CH, 2 * H), jnp.float32),
                       pltpu.VMEM((CH, 2 * H), jnp.float32),
                       pltpu.VMEM((CH, 2 * H), jnp.float32),
                       pltpu.VMEM((EW + _L,), jnp.float32),
                       pltpu.SemaphoreType.DMA,
                       pltpu.SemaphoreType.DMA,
                       pltpu.SemaphoreType.DMA,
                       pltpu.SemaphoreType.DMA],
        compiler_params=pltpu.CompilerParams(needs_layout_passes=False),
    )(t_tab, no2.reshape(NP), ni2.reshape(NP), edge_index[0], edge_index[1])

    BE = 3200
    adj = pl.pallas_call(
        _mlp_body,
        grid=(E // BE,),
        in_specs=[
            pl.BlockSpec((BE, 2 * H), lambda i: (i, 0)),
            pl.BlockSpec((1, BE), lambda i: (0, i)),
            pl.BlockSpec((1, BE), lambda i: (0, i)),
            pl.BlockSpec((2 * H, H), lambda i: (0, 0)),
            pl.BlockSpec((H, 1), lambda i: (0, 0)),
            pl.BlockSpec((H, 1), lambda i: (0, 0)),
            pl.BlockSpec((H, 1), lambda i: (0, 0)),
            pl.BlockSpec((H, 1), lambda i: (0, 0)),
            pl.BlockSpec((H, H2), lambda i: (0, 0)),
            pl.BlockSpec((H2, 1), lambda i: (0, 0)),
            pl.BlockSpec((H2, 1), lambda i: (0, 0)),
            pl.BlockSpec((H2, 1), lambda i: (0, 0)),
            pl.BlockSpec((H2, 1), lambda i: (0, 0)),
            pl.BlockSpec((H2, 1), lambda i: (0, 0)),
            pl.BlockSpec((1, 1), lambda i: (0, 0)),
        ],
        out_specs=pl.BlockSpec((1, BE), lambda i: (0, i)),
        out_shape=jax.ShapeDtypeStruct((1, E), jnp.float32),
    )(g_rows, gate.reshape(1, E), en.reshape(1, E),
      Wc, b1c.reshape(H, 1), vv1, g1.reshape(H, 1), be1.reshape(H, 1),
      W2c, b2c.reshape(H2, 1), vv2, g2.reshape(H2, 1), be2.reshape(H2, 1),
      W3, b3.reshape(1, 1))

    return adj.reshape(E)


# speedup vs baseline: 1.1516x; 1.1008x over previous
import functools

import jax
import jax.numpy as jnp
from jax import lax
from jax.experimental import pallas as pl
from jax.experimental.pallas import tpu as pltpu
from jax.experimental.pallas import tpu_sc as plsc

_EPS = 1e-5
_L = 16


def _deg_body(NP, EW, NC, src_hbm, dst_hbm, ho_hbm, hi_hbm, idx_v, hist_o, hist_i):
    wid = lax.axis_index("s") * NC + lax.axis_index("c")
    zeros = jnp.zeros((_L,), jnp.float32)
    ones = jnp.full((_L,), 1.0, jnp.float32)

    def zero_body(i, carry):
        hist_o[pl.ds(i * _L, _L)] = zeros
        hist_i[pl.ds(i * _L, _L)] = zeros
        return carry

    lax.fori_loop(0, NP // _L, zero_body, 0)

    base = wid * EW
    pltpu.sync_copy(src_hbm.at[pl.ds(base, EW)], idx_v)

    def add_o(i, carry):
        plsc.addupdate_scatter(hist_o, [idx_v[pl.ds(i * _L, _L)]], ones)
        return carry

    lax.fori_loop(0, EW // _L, add_o, 0)

    pltpu.sync_copy(dst_hbm.at[pl.ds(base, EW)], idx_v)

    def add_i(i, carry):
        plsc.addupdate_scatter(hist_i, [idx_v[pl.ds(i * _L, _L)]], ones)
        return carry

    lax.fori_loop(0, EW // _L, add_i, 0)

    pltpu.sync_copy(hist_o, ho_hbm.at[wid])
    pltpu.sync_copy(hist_i, hi_hbm.at[wid])


def _proj_body(x_ref, w1c_ref, ho_ref, hi_ref, t_ref, no_ref, ni_ref):
    t_ref[...] = jnp.dot(x_ref[...], w1c_ref[...],
                         preferred_element_type=jnp.float32)
    no_ref[...] = lax.rsqrt(jnp.maximum(jnp.sum(ho_ref[...], axis=0), 1.0))
    ni_ref[...] = lax.rsqrt(jnp.maximum(jnp.sum(hi_ref[...], axis=0), 1.0))


def _gather_body(NP, EW, NC, CH, H,
                 t_hbm, no_hbm, ni_hbm, src_hbm, dst_hbm,
                 g_hbm, en_hbm,
                 sidx, didx, no_tab, ni_tab,
                 ts0, td0, ts1, td1, gb0, gb1, enbuf,
                 sem0, sem1, sems0, sems1):
    wid = lax.axis_index("s") * NC + lax.axis_index("c")
    base = wid * EW
    pltpu.sync_copy(no_hbm, no_tab)
    pltpu.sync_copy(ni_hbm, ni_tab)
    pltpu.sync_copy(src_hbm.at[pl.ds(base, EW)], sidx.at[pl.ds(0, EW)])
    pltpu.sync_copy(dst_hbm.at[pl.ds(base, EW)], didx.at[pl.ds(0, EW)])
    zeros16 = jnp.zeros((_L,), jnp.int32)
    sidx[pl.ds(EW, _L)] = zeros16
    didx[pl.ds(EW, _L)] = zeros16

    nfull = EW // CH
    last = nfull - 1

    def issue(c, ts, td, sem):
        off = c * CH
        pltpu.async_copy(t_hbm.at[sidx.at[pl.ds(off, CH)]], ts, sem)
        pltpu.async_copy(t_hbm.at[didx.at[pl.ds(off, CH)]], td, sem)

    def process(c, ts, td, gb, sem, sems, wait_store):
        off = c * CH
        pltpu.make_async_copy(t_hbm.at[sidx.at[pl.ds(off, CH)]], ts, sem).wait()
        pltpu.make_async_copy(t_hbm.at[didx.at[pl.ds(off, CH)]], td, sem).wait()
        if wait_store:
            pltpu.make_async_copy(gb, g_hbm.at[pl.ds(base + off, CH)],
                                  sems).wait()

        def row(i, carry):
            for k in range(H // _L):
                sl = pl.ds(k * _L, _L)
                sh = pl.ds(H + k * _L, _L)
                gb[i, sl] = ts[i, sl]
                gb[i, sh] = td[i, sh]
            return carry

        lax.fori_loop(0, CH, row, 0)

        for k in range(CH // _L):
            s16 = sidx[pl.ds(off + k * _L, _L)]
            d16 = didx[pl.ds(off + k * _L, _L)]
            en16 = plsc.load_gather(no_tab, [s16]) * plsc.load_gather(ni_tab, [d16])
            enbuf[pl.ds(off + k * _L, _L)] = en16

        pltpu.async_copy(gb, g_hbm.at[pl.ds(base + off, CH)], sems)

    issue(0, ts0, td0, sem0)
    issue(1, ts1, td1, sem1)
    process(0, ts0, td0, gb0, sem0, sems0, False)
    issue(2, ts0, td0, sem0)
    process(1, ts1, td1, gb1, sem1, sems1, False)

    npair = (nfull + 1) // 2 + 1

    def pair(cc, carry):
        c1 = jnp.minimum(2 * cc + 1, last)
        c2 = jnp.minimum(2 * cc + 2, last)
        c0 = jnp.minimum(2 * cc, last)
        issue(c1, ts1, td1, sem1)
        process(c0, ts0, td0, gb0, sem0, sems0, True)
        issue(c2, ts0, td0, sem0)
        process(c1, ts1, td1, gb1, sem1, sems1, True)
        return carry

    lax.fori_loop(1, npair, pair, 0)
    process(last, ts0, td0, gb0, sem0, sems0, True)
    pltpu.make_async_copy(gb0, g_hbm.at[pl.ds(base + last * CH, CH)],
                          sems0).wait()
    pltpu.make_async_copy(gb1, g_hbm.at[pl.ds(base + last * CH, CH)],
                          sems1).wait()

    tail = EW - nfull * CH
    if tail:
        toff = nfull * CH
        pltpu.async_copy(t_hbm.at[sidx.at[pl.ds(toff, tail)]],
                         ts0.at[pl.ds(0, tail)], sem0).wait()
        pltpu.async_copy(t_hbm.at[didx.at[pl.ds(toff, tail)]],
                         td0.at[pl.ds(0, tail)], sem0).wait()

        def trow(i, carry):
            for k in range(H // _L):
                sl = pl.ds(k * _L, _L)
                sh = pl.ds(H + k * _L, _L)
                gb0[i, sl] = ts0[i, sl]
                gb0[i, sh] = td0[i, sh]
            return carry

        lax.fori_loop(0, tail, trow, 0)
        for k in range(-(-tail // _L)):
            s16 = sidx[pl.ds(toff + k * _L, _L)]
            d16 = didx[pl.ds(toff + k * _L, _L)]
            en16 = plsc.load_gather(no_tab, [s16]) * plsc.load_gather(ni_tab, [d16])
            enbuf[pl.ds(toff + k * _L, _L)] = en16
        pltpu.sync_copy(gb0.at[pl.ds(0, tail)],
                        g_hbm.at[pl.ds(base + toff, tail)])

    pltpu.sync_copy(enbuf.at[pl.ds(0, EW)], en_hbm.at[pl.ds(base, EW)])


def _tdot(a, b):
    return lax.dot_general(a, b, (((0,), (0,)), ((), ())),
                           preferred_element_type=jnp.float32)


def _mlp_body(g_ref, gate_ref, en_ref, wc_ref, b1c_ref, vv1_ref, g1_ref,
              be1_ref, w2c_ref, b2c_ref, vv2_ref, g2_ref, be2_ref, w3_ref,
              b3_ref, adj_ref):
    g = g_ref[...]
    c = lax.dot_general(wc_ref[...], g, (((0,), (1,)), ((), ())),
                        preferred_element_type=jnp.float32) + b1c_ref[...]
    v = _tdot(vv1_ref[...], c * c)
    h = c * lax.rsqrt(v + _EPS) * g1_ref[...] + be1_ref[...]
    h = jnp.maximum(h, 0.0)
    c2 = _tdot(w2c_ref[...], h) + b2c_ref[...]
    v2 = _tdot(vv2_ref[...], c2 * c2)
    h2 = c2 * lax.rsqrt(v2 + _EPS) * g2_ref[...] + be2_ref[...]
    h2 = jnp.maximum(h2, 0.0)
    w = _tdot(w3_ref[...], h2) + b3_ref[...]
    adj_ref[...] = jax.nn.sigmoid(gate_ref[...] + w) * en_ref[...]


def kernel(node_embeddings, edge_index, W1, b1, g1, be1, W2, b2, g2, be2, W3, b3):
    N, D = node_embeddings.shape
    E = edge_index.shape[1]
    H = W1.shape[1]
    H2 = W2.shape[1]
    info = plsc.get_sparse_core_info()
    NC, NS = info.num_cores, info.num_subcores
    NW = NC * NS
    EW = E // NW
    RB = 1024
    NP = -(-N // RB) * RB
    CH = 96

    noise = jax.random.uniform(jax.random.key(42), (E,), dtype=jnp.float32,
                               minval=1e-6, maxval=1.0 - 1e-6)
    gate = jnp.log(noise) - jnp.log(1.0 - noise)

    W1c = jnp.concatenate([W1[:D], W1[D:]], axis=1)
    C1 = jnp.eye(H, dtype=jnp.float32) - jnp.full((H, H), 1.0 / H, jnp.float32)
    Wc = jnp.concatenate([C1, C1], axis=0)
    b1c = (b1 - jnp.mean(b1)).reshape(1, H)
    vv1 = jnp.full((H, 1), 1.0 / H, jnp.float32)
    C2 = jnp.eye(H2, dtype=jnp.float32) - jnp.full((H2, H2), 1.0 / H2, jnp.float32)
    W2c = W2 @ C2
    b2c = (b2 - jnp.mean(b2)).reshape(1, H2)
    vv2 = jnp.full((H2, 1), 1.0 / H2, jnp.float32)

    mesh = plsc.VectorSubcoreMesh(core_axis_name="c", subcore_axis_name="s")
    ho, hi = pl.kernel(
        functools.partial(_deg_body, NP, EW, NC),
        out_type=[jax.ShapeDtypeStruct((NW, NP), jnp.float32),
                  jax.ShapeDtypeStruct((NW, NP), jnp.float32)],
        mesh=mesh,
        scratch_types=[pltpu.VMEM((EW,), jnp.int32),
                       pltpu.VMEM((NP,), jnp.float32),
                       pltpu.VMEM((NP,), jnp.float32)],
        compiler_params=pltpu.CompilerParams(needs_layout_passes=False),
    )(edge_index[0], edge_index[1])

    t_tab, no2, ni2 = pl.pallas_call(
        _proj_body,
        grid=(NP // RB,),
        in_specs=[
            pl.BlockSpec((RB, D), lambda i: (i, 0)),
            pl.BlockSpec((D, 2 * H), lambda i: (0, 0)),
            pl.BlockSpec((NW, RB // 128, 128), lambda i: (0, i, 0)),
            pl.BlockSpec((NW, RB // 128, 128), lambda i: (0, i, 0)),
        ],
        out_specs=[
            pl.BlockSpec((RB, 2 * H), lambda i: (i, 0)),
            pl.BlockSpec((RB // 128, 128), lambda i: (i, 0)),
            pl.BlockSpec((RB // 128, 128), lambda i: (i, 0)),
        ],
        out_shape=[jax.ShapeDtypeStruct((NP, 2 * H), jnp.float32),
                   jax.ShapeDtypeStruct((NP // 128, 128), jnp.float32),
                   jax.ShapeDtypeStruct((NP // 128, 128), jnp.float32)],
    )(node_embeddings, W1c,
      ho.reshape(NW, NP // 128, 128), hi.reshape(NW, NP // 128, 128))

    g_rows, en = pl.kernel(
        functools.partial(_gather_body, NP, EW, NC, CH, H),
        out_type=[jax.ShapeDtypeStruct((E, 2 * H), jnp.float32),
                  jax.ShapeDtypeStruct((E,), jnp.float32)],
        mesh=mesh,
        scratch_types=[pltpu.VMEM((EW + _L,), jnp.int32),
                       pltpu.VMEM((EW + _L,), jnp.int32),
                       pltpu.VMEM((NP,), jnp.float32),
                       pltpu.VMEM((NP,), jnp.float32),
                       pltpu.VMEM((CH, 2 * H), jnp.float32),
                       pltpu.VMEM((CH, 2 * H), jnp.float32),
                       pltpu.VMEM((CH, 2 * H), jnp.float32),
                       pltpu.VMEM((CH, 2 * H), jnp.float32),
                       pltpu.VMEM((CH, 2 * H), jnp.float32),
                       pltpu.VMEM((CH, 2 * H), jnp.float32),
                       pltpu.VMEM((EW + _L,), jnp.float32),
                       pltpu.SemaphoreType.DMA,
                       pltpu.SemaphoreType.DMA,
                       pltpu.SemaphoreType.DMA,
                       pltpu.SemaphoreType.DMA],
        compiler_params=pltpu.CompilerParams(needs_layout_passes=False),
    )(t_tab, no2.reshape(NP), ni2.reshape(NP), edge_index[0], edge_index[1])

    BE = 6400
    adj = pl.pallas_call(
        _mlp_body,
        grid=(E // BE,),
        in_specs=[
            pl.BlockSpec((BE, 2 * H), lambda i: (i, 0)),
            pl.BlockSpec((1, BE), lambda i: (0, i)),
            pl.BlockSpec((1, BE), lambda i: (0, i)),
            pl.BlockSpec((2 * H, H), lambda i: (0, 0)),
            pl.BlockSpec((H, 1), lambda i: (0, 0)),
            pl.BlockSpec((H, 1), lambda i: (0, 0)),
            pl.BlockSpec((H, 1), lambda i: (0, 0)),
            pl.BlockSpec((H, 1), lambda i: (0, 0)),
            pl.BlockSpec((H, H2), lambda i: (0, 0)),
            pl.BlockSpec((H2, 1), lambda i: (0, 0)),
            pl.BlockSpec((H2, 1), lambda i: (0, 0)),
            pl.BlockSpec((H2, 1), lambda i: (0, 0)),
            pl.BlockSpec((H2, 1), lambda i: (0, 0)),
            pl.BlockSpec((H2, 1), lambda i: (0, 0)),
            pl.BlockSpec((1, 1), lambda i: (0, 0)),
        ],
        out_specs=pl.BlockSpec((1, BE), lambda i: (0, i)),
        out_shape=jax.ShapeDtypeStruct((1, E), jnp.float32),
    )(g_rows, gate.reshape(1, E), en.reshape(1, E),
      Wc, b1c.reshape(H, 1), vv1, g1.reshape(H, 1), be1.reshape(H, 1),
      W2c, b2c.reshape(H2, 1), vv2, g2.reshape(H2, 1), be2.reshape(H2, 1),
      W3, b3.reshape(1, 1))

    return adj.reshape(E)
